# baseline (speedup 1.0000x reference)
import functools

import jax
import jax.numpy as jnp
from jax import lax
from jax.experimental import pallas as pl
from jax.experimental.pallas import tpu as pltpu
from jax.experimental.pallas import tpu_sc as plsc

BATCH = 4096
NUM_POS = 77
EMBED_DIM = 768
LANES = 16
NUM_CORES = 2
NUM_SUBCORES = 16
NUM_WORKERS = NUM_CORES * NUM_SUBCORES
BE_PER_WORKER = BATCH // NUM_WORKERS
COLS = EMBED_DIM // LANES


def _body(idx_hbm, table_hbm, pos_hbm, out_hbm, idx_v, pos_v, buf, sem):
    wid = lax.axis_index("s") * NUM_CORES + lax.axis_index("c")
    base_be = wid * BE_PER_WORKER

    pltpu.sync_copy(idx_hbm.at[pl.ds(base_be, BE_PER_WORKER)], idx_v)
    pltpu.sync_copy(pos_hbm, pos_v)

    def elem_step(e, carry):
        pltpu.async_copy(table_hbm.at[idx_v.at[e]], buf, sem).wait()

        def row_step(r, c2):
            for c in range(COLS):
                sl = pl.ds(c * LANES, LANES)
                buf[r, sl] = buf[r, sl] + pos_v[r, sl]
            return c2

        lax.fori_loop(0, NUM_POS, row_step, 0, unroll=False)

        pltpu.sync_copy(buf, out_hbm.at[base_be + e])
        return carry

    lax.fori_loop(0, BE_PER_WORKER, elem_step, 0, unroll=False)


@functools.partial(jax.jit, static_argnums=())
def _embed(idx, token_table, pos_table):
    mesh = plsc.VectorSubcoreMesh(core_axis_name="c", subcore_axis_name="s")
    fn = pl.kernel(
        _body,
        out_type=jax.ShapeDtypeStruct((BATCH, NUM_POS, EMBED_DIM), jnp.float32),
        mesh=mesh,
        compiler_params=pltpu.CompilerParams(use_tc_tiling_on_sc=False),
        scratch_types=[
            pltpu.VMEM((BE_PER_WORKER, NUM_POS), jnp.int32),
            pltpu.VMEM((NUM_POS, EMBED_DIM), jnp.float32),
            pltpu.VMEM((NUM_POS, EMBED_DIM), jnp.float32),
            pltpu.SemaphoreType.DMA,
        ],
    )
    return fn(idx, token_table, pos_table)


def kernel(input_tokens, token_table, pos_table):
    idx = input_tokens.astype(jnp.int32)
    return _embed(idx, token_table, pos_table)
